# baseline (speedup 1.0000x reference)
import jax
import jax.numpy as jnp
from jax import lax
from jax.experimental import pallas as pl
from jax.experimental.pallas import tpu as pltpu
from jax.experimental.pallas import tpu_sc as plsc

IN_DIM = 64
WIDTH = 64
K = 8
RADIUS = 0.1

QB = 128
PD = 16
GW = 128
NG = 128
TD = 128

NC = 2
NS = 16
NW = NC * NS
CH = 128


def _knn_a1_body(x_ref, p_ref, d2_ref, grp_ref):
    x = x_ref[...]
    p = p_ref[...]
    xn = jnp.sum(x * x, axis=1, keepdims=True)
    pn = jnp.sum(p * p, axis=0, keepdims=True)
    dot = lax.dot_general(x, p, (((1,), (0,)), ((), ())),
                          preferred_element_type=jnp.float32)
    d2 = xn + pn - 2.0 * dot
    d2_ref[...] = d2
    cols = [jnp.min(lax.slice_in_dim(d2, g * GW, (g + 1) * GW, axis=1),
                    axis=1, keepdims=True) for g in range(NG)]
    c = jnp.concatenate(cols, axis=1)
    piota = lax.broadcasted_iota(jnp.int32, c.shape, 1).astype(jnp.float32)
    grps = []
    for k in range(K):
        mv = jnp.min(c, axis=1, keepdims=True)
        im = jnp.min(jnp.where(c <= mv, piota, 3e7), axis=1, keepdims=True)
        grps.append(im)
        if k < K - 1:
            c = jnp.where(piota == im, 1e30, c)
    grp_ref[...] = jnp.concatenate(grps, axis=1).astype(jnp.int32)


def _knn_a2_body(cand_ref, grp_ref, idx_ref, w_ref):
    cand = cand_ref[...].reshape(QB, K * GW)
    grp = grp_ref[...].astype(jnp.float32)
    gb = jnp.broadcast_to(grp[:, :, None], (QB, K, GW)).reshape(QB, K * GW)
    l = lax.broadcasted_iota(jnp.int32, (QB, K * GW), 1)
    lmod = (l & (GW - 1)).astype(jnp.float32)
    gi = gb * float(GW) + lmod
    dists = []
    idxs = []
    for k in range(K):
        mv = jnp.min(cand, axis=1, keepdims=True)
        im = jnp.min(jnp.where(cand <= mv, gi, 3e7), axis=1, keepdims=True)
        dists.append(mv)
        idxs.append(im)
        if k < K - 1:
            cand = jnp.where(gi == im, 1e30, cand)
    d2k = jnp.concatenate(dists, axis=1)
    idx = jnp.concatenate(idxs, axis=1).astype(jnp.int32)
    dist = jnp.sqrt(jnp.maximum(d2k, 1e-12))
    valid = (dist < RADIUS).astype(jnp.float32)
    w = valid / (dist + 1e-8)
    wts = w / (jnp.sum(w, axis=1, keepdims=True) + 1e-8)
    idx_ref[...] = idx
    w_ref[...] = wts


def _sc_gather_body(idx_hbm, tab_hbm, out_hbm, idx_v, rows_v, sem):
    nch = idx_hbm.shape[0] // NW
    half = nch // 2
    b_per_w = nch * CH
    wid = lax.axis_index("s") * NC + lax.axis_index("c")
    base = wid * nch
    pltpu.sync_copy(idx_hbm.at[pl.ds(base, nch)], idx_v)
    for h in range(2):
        copies = []
        for j in range(half):
            copies.append(pltpu.async_copy(
                tab_hbm.at[idx_v.at[h * half + j]],
                rows_v.at[pl.ds(j * CH, CH)], sem))
        for c in copies:
            c.wait()
        pltpu.sync_copy(
            rows_v,
            out_hbm.at[pl.ds(wid * b_per_w + h * half * CH, half * CH)])


def _mlp_body(g_ref, x_ref, w_ref, w0f_ref, w0p_ref, b0_ref,
              w1_ref, b1_ref, wc_ref, bc_ref, wf_ref, bf_ref, out_ref):
    n = QB * K
    g = g_ref[...].reshape(n, TD)
    gf = g[:, :IN_DIM]
    gp = g[:, IN_DIM:IN_DIM + PD]
    x = x_ref[...]
    xr = jnp.broadcast_to(x[:, None, :], (QB, K, PD)).reshape(n, PD)
    rel = xr - gp
    h = gf @ w0f_ref[...] + rel @ w0p_ref[...] + b0_ref[...]
    h = jnp.maximum(h, 0.0)
    h = jnp.maximum(h @ w1_ref[...] + b1_ref[...], 0.0)
    s = jnp.sum(h * wc_ref[...], axis=1, keepdims=True) + bc_ref[...]
    conf = jax.nn.sigmoid(s)
    o = h @ wf_ref[...] + bf_ref[...]
    scale = conf.reshape(QB, K, 1) * w_ref[...][:, :, None]
    out_ref[...] = jnp.sum(o.reshape(QB, K, IN_DIM) * scale, axis=1)


def kernel(xyz, pcd, feat, W0, b0, W1, b1, Wd, bd, Wc, bc, Wf, bf):
    q = xyz.shape[0]
    m = pcd.shape[0]
    f32 = jnp.float32
    b = q * K
    b_per_w = b // NW
    nch = b_per_w // CH
    sc_mesh = plsc.VectorSubcoreMesh(
        core_axis_name="c", subcore_axis_name="s",
        num_cores=NC, num_subcores=NS)

    def sc_gather(idx2d, table):
        return pl.kernel(
            _sc_gather_body,
            out_type=jax.ShapeDtypeStruct((b, TD), f32),
            mesh=sc_mesh,
            scratch_types=[
                pltpu.VMEM((nch, CH), jnp.int32),
                pltpu.VMEM((b_per_w // 2, TD), f32),
                pltpu.SemaphoreType.DMA,
            ],
        )(idx2d, table)

    x8 = jnp.pad(xyz, ((0, 0), (0, 5)))
    pt = jnp.pad(pcd, ((0, 0), (0, 5))).T
    d2g, grp = pl.pallas_call(
        _knn_a1_body,
        grid=(q // QB,),
        in_specs=[
            pl.BlockSpec((QB, 8), lambda i: (i, 0)),
            pl.BlockSpec((8, m), lambda i: (0, 0)),
        ],
        out_specs=[
            pl.BlockSpec((QB, m), lambda i: (i, 0)),
            pl.BlockSpec((QB, K), lambda i: (i, 0)),
        ],
        out_shape=[
            jax.ShapeDtypeStruct((q, m), f32),
            jax.ShapeDtypeStruct((q, K), jnp.int32),
        ],
    )(x8, pt)

    rowidx = jnp.arange(q, dtype=jnp.int32)[:, None] * NG + grp
    cand = sc_gather(rowidx.reshape(b // CH, CH), d2g.reshape(q * NG, GW))

    idx, wts = pl.pallas_call(
        _knn_a2_body,
        grid=(q // QB,),
        in_specs=[
            pl.BlockSpec((QB, K, GW), lambda i: (i, 0, 0)),
            pl.BlockSpec((QB, K), lambda i: (i, 0)),
        ],
        out_specs=[
            pl.BlockSpec((QB, K), lambda i: (i, 0)),
            pl.BlockSpec((QB, K), lambda i: (i, 0)),
        ],
        out_shape=[
            jax.ShapeDtypeStruct((q, K), jnp.int32),
            jax.ShapeDtypeStruct((q, K), f32),
        ],
    )(cand.reshape(q, K, GW), grp)

    table = jnp.pad(jnp.concatenate([feat, pcd], axis=1),
                    ((0, 0), (0, TD - IN_DIM - 3)))
    g = sc_gather(idx.reshape(b // CH, CH), table)

    x16 = jnp.pad(xyz, ((0, 0), (0, PD - 3)))
    w0f = W0[:IN_DIM]
    w0p = jnp.pad(W0[IN_DIM:], ((0, PD - 3), (0, 0)))
    full = lambda shape: pl.BlockSpec(shape, lambda i: tuple(0 for _ in shape))
    out = pl.pallas_call(
        _mlp_body,
        grid=(q // QB,),
        in_specs=[
            pl.BlockSpec((QB, K, TD), lambda i: (i, 0, 0)),
            pl.BlockSpec((QB, PD), lambda i: (i, 0)),
            pl.BlockSpec((QB, K), lambda i: (i, 0)),
            full((IN_DIM, WIDTH)),
            full((PD, WIDTH)),
            full((1, WIDTH)),
            full((WIDTH, WIDTH)),
            full((1, WIDTH)),
            full((1, WIDTH)),
            full((1, 1)),
            full((WIDTH, IN_DIM)),
            full((1, IN_DIM)),
        ],
        out_specs=pl.BlockSpec((QB, IN_DIM), lambda i: (i, 0)),
        out_shape=jax.ShapeDtypeStruct((q, IN_DIM), f32),
    )(g.reshape(q, K, TD), x16, wts,
      w0f, w0p, b0.reshape(1, WIDTH), W1, b1.reshape(1, WIDTH),
      Wc.reshape(1, WIDTH), bc.reshape(1, 1), Wf, bf.reshape(1, IN_DIM))
    return out


# speedup vs baseline: 2.0002x; 2.0002x over previous
import jax
import jax.numpy as jnp
from jax import lax
from jax.experimental import pallas as pl
from jax.experimental.pallas import tpu as pltpu
from jax.experimental.pallas import tpu_sc as plsc

IN_DIM = 64
WIDTH = 64
K = 8
RADIUS = 0.1

QB = 128
PD = 16
GW = 128
NG = 128
TD = 128

NC = 2
NS = 16
NW = NC * NS
CH = 128


def _knn_a1_body(x_ref, p_ref, d2_ref, grp_ref):
    x = x_ref[...]
    p = p_ref[...]
    xn = jnp.sum(x * x, axis=1, keepdims=True)
    pn = jnp.sum(p * p, axis=0, keepdims=True)
    dot = lax.dot_general(x, p, (((1,), (0,)), ((), ())),
                          preferred_element_type=jnp.float32)
    d2 = xn + pn - 2.0 * dot
    cols = []
    for g in range(NG):
        s = lax.slice_in_dim(d2, g * GW, (g + 1) * GW, axis=1)
        d2_ref[g] = s
        cols.append(jnp.min(s, axis=1, keepdims=True))
    c = jnp.concatenate(cols, axis=1)
    piota = lax.broadcasted_iota(jnp.int32, c.shape, 1).astype(jnp.float32)
    grps = []
    for k in range(K):
        mv = jnp.min(c, axis=1, keepdims=True)
        im = jnp.min(jnp.where(c <= mv, piota, 3e7), axis=1, keepdims=True)
        grps.append(im)
        if k < K - 1:
            c = jnp.where(piota == im, 1e30, c)
    grp_ref[...] = jnp.concatenate(grps, axis=1).astype(jnp.int32)


def _knn_a2_body(cand_ref, grp_ref, idx_ref, w_ref):
    cand = cand_ref[...].reshape(QB, K * GW)
    grp = grp_ref[...].astype(jnp.float32)
    gb = jnp.broadcast_to(grp[:, :, None], (QB, K, GW)).reshape(QB, K * GW)
    l = lax.broadcasted_iota(jnp.int32, (QB, K * GW), 1)
    lmod = (l & (GW - 1)).astype(jnp.float32)
    gi = gb * float(GW) + lmod
    dists = []
    idxs = []
    for k in range(K):
        mv = jnp.min(cand, axis=1, keepdims=True)
        im = jnp.min(jnp.where(cand <= mv, gi, 3e7), axis=1, keepdims=True)
        dists.append(mv)
        idxs.append(im)
        if k < K - 1:
            cand = jnp.where(gi == im, 1e30, cand)
    d2k = jnp.concatenate(dists, axis=1)
    idx = jnp.concatenate(idxs, axis=1).astype(jnp.int32)
    dist = jnp.sqrt(jnp.maximum(d2k, 1e-12))
    valid = (dist < RADIUS).astype(jnp.float32)
    w = valid / (dist + 1e-8)
    wts = w / (jnp.sum(w, axis=1, keepdims=True) + 1e-8)
    idx_ref[...] = idx
    w_ref[...] = wts


def _sc_gather_body(idx_hbm, tab_hbm, out_hbm, idx_v, rows_v, sem):
    nch = idx_hbm.shape[0] // NW
    half = nch // 2
    b_per_w = nch * CH
    wid = lax.axis_index("s") * NC + lax.axis_index("c")
    base = wid * nch
    pltpu.sync_copy(idx_hbm.at[pl.ds(base, nch)], idx_v)
    for h in range(2):
        copies = []
        for j in range(half):
            copies.append(pltpu.async_copy(
                tab_hbm.at[idx_v.at[h * half + j]],
                rows_v.at[pl.ds(j * CH, CH)], sem))
        for c in copies:
            c.wait()
        pltpu.sync_copy(
            rows_v,
            out_hbm.at[pl.ds(wid * b_per_w + h * half * CH, half * CH)])


def _mlp_body(g_ref, x_ref, w_ref, w0f_ref, w0p_ref, b0_ref,
              w1_ref, b1_ref, wc_ref, bc_ref, wf_ref, bf_ref, out_ref):
    n = QB * K
    g = g_ref[...].reshape(n, TD)
    gf = g[:, :IN_DIM]
    gp = g[:, IN_DIM:IN_DIM + PD]
    x = x_ref[...]
    xr = jnp.broadcast_to(x[:, None, :], (QB, K, PD)).reshape(n, PD)
    rel = xr - gp
    h = gf @ w0f_ref[...] + rel @ w0p_ref[...] + b0_ref[...]
    h = jnp.maximum(h, 0.0)
    h = jnp.maximum(h @ w1_ref[...] + b1_ref[...], 0.0)
    s = jnp.sum(h * wc_ref[...], axis=1, keepdims=True) + bc_ref[...]
    conf = jax.nn.sigmoid(s)
    o = h @ wf_ref[...] + bf_ref[...]
    scale = conf.reshape(QB, K, 1) * w_ref[...][:, :, None]
    out_ref[...] = jnp.sum(o.reshape(QB, K, IN_DIM) * scale, axis=1)


def kernel(xyz, pcd, feat, W0, b0, W1, b1, Wd, bd, Wc, bc, Wf, bf):
    q = xyz.shape[0]
    m = pcd.shape[0]
    f32 = jnp.float32
    b = q * K
    b_per_w = b // NW
    nch = b_per_w // CH
    sc_mesh = plsc.VectorSubcoreMesh(
        core_axis_name="c", subcore_axis_name="s",
        num_cores=NC, num_subcores=NS)

    def sc_gather(idx2d, table):
        return pl.kernel(
            _sc_gather_body,
            out_type=jax.ShapeDtypeStruct((b, TD), f32),
            mesh=sc_mesh,
            scratch_types=[
                pltpu.VMEM((nch, CH), jnp.int32),
                pltpu.VMEM((b_per_w // 2, TD), f32),
                pltpu.SemaphoreType.DMA,
            ],
        )(idx2d, table)

    x8 = jnp.pad(xyz, ((0, 0), (0, 5)))
    pt = jnp.pad(pcd, ((0, 0), (0, 5))).T
    d2g, grp = pl.pallas_call(
        _knn_a1_body,
        grid=(q // QB,),
        in_specs=[
            pl.BlockSpec((QB, 8), lambda i: (i, 0)),
            pl.BlockSpec((8, m), lambda i: (0, 0)),
        ],
        out_specs=[
            pl.BlockSpec((NG, QB, GW), lambda i: (0, i, 0)),
            pl.BlockSpec((QB, K), lambda i: (i, 0)),
        ],
        out_shape=[
            jax.ShapeDtypeStruct((NG, q, GW), f32),
            jax.ShapeDtypeStruct((q, K), jnp.int32),
        ],
    )(x8, pt)

    rowidx = grp * q + jnp.arange(q, dtype=jnp.int32)[:, None]
    cand = sc_gather(rowidx.reshape(b // CH, CH), d2g.reshape(NG * q, GW))

    idx, wts = pl.pallas_call(
        _knn_a2_body,
        grid=(q // QB,),
        in_specs=[
            pl.BlockSpec((QB, K, GW), lambda i: (i, 0, 0)),
            pl.BlockSpec((QB, K), lambda i: (i, 0)),
        ],
        out_specs=[
            pl.BlockSpec((QB, K), lambda i: (i, 0)),
            pl.BlockSpec((QB, K), lambda i: (i, 0)),
        ],
        out_shape=[
            jax.ShapeDtypeStruct((q, K), jnp.int32),
            jax.ShapeDtypeStruct((q, K), f32),
        ],
    )(cand.reshape(q, K, GW), grp)

    table = jnp.pad(jnp.concatenate([feat, pcd], axis=1),
                    ((0, 0), (0, TD - IN_DIM - 3)))
    g = sc_gather(idx.reshape(b // CH, CH), table)

    x16 = jnp.pad(xyz, ((0, 0), (0, PD - 3)))
    w0f = W0[:IN_DIM]
    w0p = jnp.pad(W0[IN_DIM:], ((0, PD - 3), (0, 0)))
    full = lambda shape: pl.BlockSpec(shape, lambda i: tuple(0 for _ in shape))
    out = pl.pallas_call(
        _mlp_body,
        grid=(q // QB,),
        in_specs=[
            pl.BlockSpec((QB, K, TD), lambda i: (i, 0, 0)),
            pl.BlockSpec((QB, PD), lambda i: (i, 0)),
            pl.BlockSpec((QB, K), lambda i: (i, 0)),
            full((IN_DIM, WIDTH)),
            full((PD, WIDTH)),
            full((1, WIDTH)),
            full((WIDTH, WIDTH)),
            full((1, WIDTH)),
            full((1, WIDTH)),
            full((1, 1)),
            full((WIDTH, IN_DIM)),
            full((1, IN_DIM)),
        ],
        out_specs=pl.BlockSpec((QB, IN_DIM), lambda i: (i, 0)),
        out_shape=jax.ShapeDtypeStruct((q, IN_DIM), f32),
    )(g.reshape(q, K, TD), x16, wts,
      w0f, w0p, b0.reshape(1, WIDTH), W1, b1.reshape(1, WIDTH),
      Wc.reshape(1, WIDTH), bc.reshape(1, 1), Wf, bf.reshape(1, IN_DIM))
    return out
